# baseline (speedup 1.0000x reference)
import functools

import numpy as np
import jax
import jax.numpy as jnp
from jax.experimental import pallas as pl
from jax.experimental.pallas import tpu as pltpu

_D = 18
_GS = 9
_NC = 512
_SQRT_D = np.float32(np.sqrt(np.float32(18.0)))
_QS = np.float32(np.float32(1.0) / _SQRT_D)
_ROWS = 512


def _codebook_w():
    codes = np.arange(_NC)
    gb = 2 ** np.arange(_GS - 1, -1, -1)
    cb = (((codes[:, None] // gb) % 2) * 2 - 1).astype(np.float32)
    wblk = (2.0 * (cb / _SQRT_D)).T.astype(np.float32)
    w = np.zeros((_D, 2 * _NC), np.float32)
    w[:_GS, :_NC] = wblk
    w[_GS:, _NC:] = wblk
    return jnp.asarray(w)


def _bsq_kernel(z_ref, w_ref, zq_ref, idx_ref, gidx_ref, avgp_ref,
                loss_ref, cbe_ref, acc_ref, s_ref, *, ntot):
    pid = pl.program_id(0)
    nsteps = pl.num_programs(0)

    @pl.when(pid == 0)
    def _init():
        acc_ref[...] = jnp.zeros_like(acc_ref)
        s_ref[0] = jnp.float32(0.0)
        s_ref[1] = jnp.float32(0.0)

    z = z_ref[...]
    zhat = jnp.where(z > 0, jnp.float32(1.0), jnp.float32(-1.0))
    zq = zhat * _QS
    zq_ref[...] = zq

    lane = jax.lax.broadcasted_iota(jnp.int32, (1, _D), 1)
    basis = jax.lax.shift_left(jnp.int32(1), (_D - 1) - lane).astype(jnp.float32)
    gshift = jnp.where(lane < _GS, (_GS - 1) - lane, (_D - 1) - lane)
    gbasis = jax.lax.shift_left(jnp.int32(1), gshift).astype(jnp.float32)
    m0 = (lane < _GS).astype(jnp.float32)
    m1 = 1.0 - m0

    t = (zq + 1.0) * 0.5
    idx_f = jnp.sum(t * basis, axis=1, keepdims=True)
    idx_ref[...] = idx_f.astype(jnp.int32)
    tg = t * gbasis
    g0 = jnp.sum(tg * m0, axis=1, keepdims=True)
    g1 = jnp.sum(tg * m1, axis=1, keepdims=True)
    gidx_ref[...] = jnp.concatenate([g0, g1], axis=1).astype(jnp.int32)

    logits = jnp.dot(z, w_ref[...], preferred_element_type=jnp.float32,
                     precision=jax.lax.Precision.HIGHEST)
    l0 = logits[:, :_NC]
    l1 = logits[:, _NC:]
    e0 = jnp.exp(l0 - jnp.max(l0, axis=1, keepdims=True))
    e1 = jnp.exp(l1 - jnp.max(l1, axis=1, keepdims=True))
    p0 = e0 / jnp.sum(e0, axis=1, keepdims=True)
    p1 = e1 / jnp.sum(e1, axis=1, keepdims=True)
    part = jnp.concatenate(
        [jnp.sum(p0, axis=0, keepdims=True),
         jnp.sum(p1, axis=0, keepdims=True)], axis=1)
    acc_ref[...] += part

    p = jax.nn.sigmoid(z * jnp.float32(-4.0 * float(_QS)))
    ent = -(p * jnp.log(p + 1e-8) + (1.0 - p) * jnp.log((1.0 - p) + 1e-8))
    s_ref[0] += jnp.sum(ent)
    diff = zq - z
    s_ref[1] += jnp.sum(diff * diff)

    @pl.when(pid == nsteps - 1)
    def _fin():
        inv_n = jnp.float32(1.0 / ntot)
        avg = acc_ref[...] * inv_n
        avgp = jnp.concatenate([avg[:, :_NC], avg[:, _NC:]], axis=0)
        avgp_ref[...] = avgp
        cbe = -jnp.sum(avgp * jnp.log(avgp + 1e-8))
        cbe_ref[...] = jnp.reshape(cbe, (1, 1))
        pse = s_ref[0] * inv_n
        commit = 0.25 * (s_ref[1] * inv_n)
        loss_ref[...] = jnp.reshape(commit + pse - cbe, (1, 1))


def kernel(z):
    b, s, d = z.shape
    n = b * s
    zf = z.reshape(n, d)
    w = _codebook_w()
    grid = n // _ROWS
    outs = pl.pallas_call(
        functools.partial(_bsq_kernel, ntot=float(n)),
        grid=(grid,),
        in_specs=[
            pl.BlockSpec((_ROWS, d), lambda i: (i, 0)),
            pl.BlockSpec((_D, 2 * _NC), lambda i: (0, 0)),
        ],
        out_specs=[
            pl.BlockSpec((_ROWS, d), lambda i: (i, 0)),
            pl.BlockSpec((_ROWS, 1), lambda i: (i, 0)),
            pl.BlockSpec((_ROWS, 2), lambda i: (i, 0)),
            pl.BlockSpec((2, _NC), lambda i: (0, 0)),
            pl.BlockSpec((1, 1), lambda i: (0, 0)),
            pl.BlockSpec((1, 1), lambda i: (0, 0)),
        ],
        out_shape=[
            jax.ShapeDtypeStruct((n, d), jnp.float32),
            jax.ShapeDtypeStruct((n, 1), jnp.int32),
            jax.ShapeDtypeStruct((n, 2), jnp.int32),
            jax.ShapeDtypeStruct((2, _NC), jnp.float32),
            jax.ShapeDtypeStruct((1, 1), jnp.float32),
            jax.ShapeDtypeStruct((1, 1), jnp.float32),
        ],
        scratch_shapes=[
            pltpu.VMEM((1, 2 * _NC), jnp.float32),
            pltpu.SMEM((2,), jnp.float32),
        ],
        compiler_params=pltpu.CompilerParams(
            dimension_semantics=("arbitrary",)),
    )(zf, w)
    zq, idx, gidx, avgp, loss, cbe = outs
    zq = zq.reshape(b, s, d)
    indices = idx.reshape(b, s).astype(jnp.int64)
    group_indices = gidx.reshape(b, s, 2).astype(jnp.int64)
    return (zq, loss[0, 0], cbe[0, 0], indices, group_indices, avgp)


# speedup vs baseline: 1.6145x; 1.6145x over previous
import functools

import numpy as np
import jax
import jax.numpy as jnp
from jax.experimental import pallas as pl
from jax.experimental.pallas import tpu as pltpu

_D = 18
_GS = 9
_NC = 512
_SQRT_D = np.float32(np.sqrt(np.float32(18.0)))
_QS = np.float32(np.float32(1.0) / _SQRT_D)
_ROWS = 512


def _codebook_w():
    codes = np.arange(_NC)
    gb = 2 ** np.arange(_GS - 1, -1, -1)
    cb = (((codes[:, None] // gb) % 2) * 2 - 1).astype(np.float32)
    w = np.zeros((_D, 2 * _NC), np.float32)
    w[:_GS, :_NC] = cb.T
    w[_GS:, _NC:] = cb.T
    return jnp.asarray(np.concatenate([w, w], axis=0))


def _bsq_kernel(z_ref, w_ref, zq_ref, idx_ref, gidx_ref, avgp_ref,
                loss_ref, cbe_ref, acc_ref, s_ref, *, ntot):
    pid = pl.program_id(0)
    nsteps = pl.num_programs(0)

    @pl.when(pid == 0)
    def _init():
        acc_ref[...] = jnp.zeros_like(acc_ref)
        s_ref[0] = jnp.float32(0.0)
        s_ref[1] = jnp.float32(0.0)

    z = z_ref[...]
    zhat = jnp.where(z > 0, jnp.float32(1.0), jnp.float32(-1.0))
    zq = zhat * _QS
    zq_ref[...] = zq

    lane = jax.lax.broadcasted_iota(jnp.int32, (1, _D), 1)
    basis = jax.lax.shift_left(jnp.int32(1), (_D - 1) - lane).astype(jnp.float32)
    gshift = jnp.where(lane < _GS, (_GS - 1) - lane, (_D - 1) - lane)
    gbasis = jax.lax.shift_left(jnp.int32(1), gshift).astype(jnp.float32)
    m0 = (lane < _GS).astype(jnp.float32)
    m1 = 1.0 - m0

    t = (zq + 1.0) * 0.5
    idx_f = jnp.sum(t * basis, axis=1, keepdims=True)
    idx_ref[...] = idx_f.astype(jnp.int32)
    tg = t * gbasis
    g0 = jnp.sum(tg * m0, axis=1, keepdims=True)
    g1 = jnp.sum(tg * m1, axis=1, keepdims=True)
    gidx_ref[...] = jnp.concatenate([g0, g1], axis=1).astype(jnp.int32)

    cz = z * jnp.float32(2.0 * float(_QS))
    hi = cz.astype(jnp.bfloat16).astype(jnp.float32)
    lo = cz - hi
    x = jnp.concatenate([hi, lo], axis=1)
    logits = jnp.dot(x, w_ref[...], preferred_element_type=jnp.float32)
    e0 = jnp.exp(logits[:, :_NC])
    e1 = jnp.exp(logits[:, _NC:])
    p0 = e0 / jnp.sum(e0, axis=1, keepdims=True)
    p1 = e1 / jnp.sum(e1, axis=1, keepdims=True)
    part = jnp.concatenate(
        [jnp.sum(p0, axis=0, keepdims=True),
         jnp.sum(p1, axis=0, keepdims=True)], axis=1)
    acc_ref[...] += part

    p = jax.nn.sigmoid(z * jnp.float32(-4.0 * float(_QS)))
    ent = -(p * jnp.log(p + 1e-8) + (1.0 - p) * jnp.log((1.0 - p) + 1e-8))
    s_ref[0] += jnp.sum(ent)
    diff = zq - z
    s_ref[1] += jnp.sum(diff * diff)

    @pl.when(pid == nsteps - 1)
    def _fin():
        inv_n = jnp.float32(1.0 / ntot)
        avg = acc_ref[...] * inv_n
        avgp = jnp.concatenate([avg[:, :_NC], avg[:, _NC:]], axis=0)
        avgp_ref[...] = avgp
        cbe = -jnp.sum(avgp * jnp.log(avgp + 1e-8))
        cbe_ref[...] = jnp.reshape(cbe, (1, 1))
        pse = s_ref[0] * inv_n
        commit = 0.25 * (s_ref[1] * inv_n)
        loss_ref[...] = jnp.reshape(commit + pse - cbe, (1, 1))


def kernel(z):
    b, s, d = z.shape
    n = b * s
    zf = z.reshape(n, d)
    w = _codebook_w()
    grid = n // _ROWS
    outs = pl.pallas_call(
        functools.partial(_bsq_kernel, ntot=float(n)),
        grid=(grid,),
        in_specs=[
            pl.BlockSpec((_ROWS, d), lambda i: (i, 0)),
            pl.BlockSpec((2 * _D, 2 * _NC), lambda i: (0, 0)),
        ],
        out_specs=[
            pl.BlockSpec((_ROWS, d), lambda i: (i, 0)),
            pl.BlockSpec((_ROWS, 1), lambda i: (i, 0)),
            pl.BlockSpec((_ROWS, 2), lambda i: (i, 0)),
            pl.BlockSpec((2, _NC), lambda i: (0, 0)),
            pl.BlockSpec((1, 1), lambda i: (0, 0)),
            pl.BlockSpec((1, 1), lambda i: (0, 0)),
        ],
        out_shape=[
            jax.ShapeDtypeStruct((n, d), jnp.float32),
            jax.ShapeDtypeStruct((n, 1), jnp.int32),
            jax.ShapeDtypeStruct((n, 2), jnp.int32),
            jax.ShapeDtypeStruct((2, _NC), jnp.float32),
            jax.ShapeDtypeStruct((1, 1), jnp.float32),
            jax.ShapeDtypeStruct((1, 1), jnp.float32),
        ],
        scratch_shapes=[
            pltpu.VMEM((1, 2 * _NC), jnp.float32),
            pltpu.SMEM((2,), jnp.float32),
        ],
        compiler_params=pltpu.CompilerParams(
            dimension_semantics=("arbitrary",)),
    )(zf, w)
    zq, idx, gidx, avgp, loss, cbe = outs
    zq = zq.reshape(b, s, d)
    indices = idx.reshape(b, s).astype(jnp.int64)
    group_indices = gidx.reshape(b, s, 2).astype(jnp.int64)
    return (zq, loss[0, 0], cbe[0, 0], indices, group_indices, avgp)
